# baseline (speedup 1.0000x reference)
import jax
import jax.numpy as jnp
from jax import lax
from jax.experimental import pallas as pl
from jax.experimental.pallas import tpu as pltpu
from jax.experimental.pallas import tpu_sc as plsc

_N = 10000
_E = 320000
_D = 128
_H = 128
_MAX_IT = 10

_NC = 2
_NS = 16
_NW = _NC * _NS
_CH = 128
_NPAD = 10112
_RPS = _NPAD // _NS
_NPASS = 4
_EPW = 10752
_EPP = _EPW // _NPASS
_NCHUNK = _EPP // _CH
_EPAD = _EPW * _NW
_LROWS = 128
_LMASK = _LROWS - 1

_HALF_SIZES = [10080] * 11 + [9840] * 4 + [9760]
_CUT_LIST = [0]
for _sz in _HALF_SIZES + _HALF_SIZES:
    _CUT_LIST.append(_CUT_LIST[-1] + _sz)


def _agg_body(state, srcs, slots, rmv, dstb, zeros, out,
              src_v, slot_v, rmv_v, rows_v, local, rowmap, aggr, dstb_v, sem):
    c = lax.axis_index("c")
    s = lax.axis_index("s")
    w = c * _NS + s
    pltpu.sync_copy(zeros, aggr.at[pl.ds(s * _RPS, _RPS)])
    plsc.subcore_barrier()

    def one_pass(p, carry):
        pltpu.sync_copy(zeros.at[pl.ds(0, _LROWS)], local)
        for g in range(_CH // 16):
            rowmap[0, pl.ds(g * 16, 16)] = jnp.full(
                (16,), _N + (g * 13) % (_NPAD - _N), jnp.int32)
        base = w * _EPW + p * _EPP

        def chunk(j, cc2):
            off = base + j * _CH
            pltpu.sync_copy(srcs.at[pl.ds(off, _CH)], src_v)
            pltpu.sync_copy(slots.at[pl.ds(off, _CH)], slot_v)
            pltpu.sync_copy(rmv.at[pl.ds(off, _CH)], rmv_v)
            pltpu.sync_copy(dstb.at[pl.ds(off * 16, _CH * 16)], dstb_v)
            pltpu.async_copy(state.at[src_v], rows_v, sem).wait()

            def edge(e, cc):
                slot = dstb_v[pl.ds(e * 16, 16)]
                for cg in range(_H // 16):
                    cols = lax.iota(jnp.int32, 16) + (cg * 16)
                    vals = rows_v[e, pl.ds(cg * 16, 16)]
                    plsc.addupdate_scatter(local, [slot, cols], vals)
                return cc

            lax.fori_loop(0, _CH, edge, 0)
            for g in range(_CH // 16):
                sl = slot_v[pl.ds(g * 16, 16)]
                mvec = rmv_v[pl.ds(g * 16, 16)]
                msk = mvec >= jnp.zeros((16,), jnp.int32)
                plsc.store_scatter(rowmap, [jnp.zeros((16,), jnp.int32), sl],
                                   mvec, mask=msk)
            return cc2

        lax.fori_loop(0, _NCHUNK, chunk, 0)
        pltpu.sync_copy(local, aggr.at[rowmap.at[0]], add=True)
        return carry

    lax.fori_loop(0, _NPASS, one_pass, 0)
    plsc.subcore_barrier()
    pltpu.sync_copy(aggr.at[pl.ds(s * _RPS, _RPS)],
                    out.at[c, pl.ds(s * _RPS, _RPS)])


_aggregate = pl.kernel(
    _agg_body,
    out_type=jax.ShapeDtypeStruct((_NC, _NPAD, _H), jnp.float32),
    mesh=plsc.VectorSubcoreMesh(core_axis_name="c", subcore_axis_name="s"),
    compiler_params=pltpu.CompilerParams(needs_layout_passes=False),
    scratch_types=[
        pltpu.VMEM((_CH,), jnp.int32),
        pltpu.VMEM((_CH,), jnp.int32),
        pltpu.VMEM((_CH,), jnp.int32),
        pltpu.VMEM((_CH, _H), jnp.float32),
        pltpu.VMEM((_LROWS, _H), jnp.float32),
        pltpu.VMEM((1, _LROWS), jnp.int32),
        pltpu.VMEM_SHARED((_NPAD, _H), jnp.float32),
        pltpu.VMEM((_CH * 16,), jnp.int32),
        pltpu.SemaphoreType.DMA,
    ],
)


def _proj_body(x_ref, wt_ref, u_ref, s_ref):
    u = jnp.dot(x_ref[...], wt_ref[...], preferred_element_type=jnp.float32)
    u_ref[...] = u
    s_ref[...] = jnp.tanh(u)


def _proj(x, wt):
    return pl.pallas_call(
        _proj_body,
        out_shape=(
            jax.ShapeDtypeStruct((_NPAD, _H), jnp.float32),
            jax.ShapeDtypeStruct((_NPAD, _H), jnp.float32),
        ),
    )(x, wt)


def _update_body(u_ref, p_ref, w_ref, o_ref):
    agg = p_ref[0] + p_ref[1]
    o_ref[...] = jnp.tanh(
        u_ref[...] + jnp.dot(agg, w_ref[...], preferred_element_type=jnp.float32))
    o_ref[pl.ds(_N, _NPAD - _N), :] = jnp.zeros((_NPAD - _N, _H), jnp.float32)


def _update(u, p, w):
    return pl.pallas_call(
        _update_body,
        out_shape=jax.ShapeDtypeStruct((_NPAD, _H), jnp.float32),
    )(u, p, w)


def _layout_edges(src, dst):
    order = jnp.argsort(dst, stable=True)
    ss = src[order]
    ds = dst[order]
    cuts = jnp.array(_CUT_LIST, jnp.int32)
    left = jnp.searchsorted(ds, ds, side="left").astype(jnp.int32)
    wstart = cuts[:-1]
    wend = cuts[1:]
    cw = [wstart]
    for p in range(1, _NPASS):
        cand = jnp.minimum(cw[-1] + _EPP, wend)
        nb = jnp.where(cand >= wend, wend,
                       jnp.maximum(left[jnp.minimum(cand, _E - 1)], cw[-1]))
        cw.append(nb.astype(jnp.int32))
    cw.append(wend)
    cwm = jnp.stack(cw)
    e = jnp.arange(_E, dtype=jnp.int32)
    w_e = jnp.searchsorted(cuts[1:], e, side="right").astype(jnp.int32)
    pa = ((e >= cwm[1][w_e]).astype(jnp.int32)
          + (e >= cwm[2][w_e]).astype(jnp.int32)
          + (e >= cwm[3][w_e]).astype(jnp.int32))
    pass_base = cwm[pa, w_e]
    newpos = w_e * _EPW + pa * _EPP + (e - pass_base)
    ar = jnp.arange(_EPAD, dtype=jnp.int32)
    src_p = (_N + ar % (_NPAD - _N)).at[newpos].set(ss)
    slotv = jnp.zeros((_EPAD,), jnp.int32).at[newpos].set(
        jnp.bitwise_and(ds, _LMASK))
    rmv = jnp.full((_EPAD,), -1, jnp.int32).at[newpos].set(ds)
    slotb = jnp.repeat(slotv, 16)
    return src_p, slotv, rmv, slotb


def kernel(edge_index, x, W_in0, W_rec0, W_in1, W_rec1):
    src = edge_index[0].astype(jnp.int32)
    dst = edge_index[1].astype(jnp.int32)
    src_p, slotv, rmv, slotb = _layout_edges(src, dst)
    zeros = jnp.zeros((_RPS, _H), jnp.float32)
    x_pad = jnp.pad(x, ((0, _NPAD - _N), (0, 0)))

    def step(u, w_rec, s):
        p = _aggregate(s, src_p, slotv, rmv, slotb, zeros)
        return _update(u, p, w_rec)

    u0, s = _proj(x_pad, W_in0.T)
    s = lax.fori_loop(0, _MAX_IT - 1, lambda i, st: step(u0, W_rec0, st), s)
    u1, s = _proj(s, W_in1.T)
    s = lax.fori_loop(0, _MAX_IT - 1, lambda i, st: step(u1, W_rec1, st), s)
    return s[:_N]
